# speedup vs baseline: 14.9658x; 14.9658x over previous
import jax
import jax.numpy as jnp
from jax import lax
from jax.experimental import pallas as pl
from jax.experimental.pallas import tpu as pltpu

_B = 128
_N = 32768
_BLOCK_ROWS = 8
_BISECT_ITERS = 24


def _sparsemax_block(x_ref, o_ref):
    z = x_ref[...]
    mx = jnp.max(z, axis=-1, keepdims=True)
    lo = mx - 1.0
    hi = mx

    def bis(_, lohi):
        lo, hi = lohi
        mid = 0.5 * (lo + hi)
        fs = jnp.sum(jnp.maximum(z - mid, 0.0), axis=-1, keepdims=True) - 1.0
        pred = fs >= 0.0
        return jnp.where(pred, mid, lo), jnp.where(pred, hi, mid)

    lo, hi = lax.fori_loop(0, _BISECT_ITERS, bis, (lo, hi))

    msk = z > lo
    csum = jnp.sum(jnp.where(msk, z, 0.0), axis=-1, keepdims=True)
    cnt = jnp.sum(jnp.where(msk, 1.0, 0.0), axis=-1, keepdims=True)
    theta = (csum - 1.0) / cnt
    o_ref[...] = jnp.maximum(z - theta, 0.0)


def kernel(inputs):
    grid = (_B // _BLOCK_ROWS,)
    return pl.pallas_call(
        _sparsemax_block,
        grid=grid,
        in_specs=[pl.BlockSpec((_BLOCK_ROWS, _N), lambda i: (i, 0))],
        out_specs=pl.BlockSpec((_BLOCK_ROWS, _N), lambda i: (i, 0)),
        out_shape=jax.ShapeDtypeStruct((_B, _N), jnp.float32),
    )(inputs)
